# Initial kernel scaffold; baseline (speedup 1.0000x reference)
import jax
import jax.numpy as jnp
from jax.experimental import pallas as pl


def kernel(features, edge_index, ln_g, ln_b, W_src, b_src, W_dst, b_dst, attn, gat_bias, Wo, bo):
    raise NotImplementedError("write your pallas kernel here")


# speedup vs baseline: 8.6467x; 8.6467x over previous
import jax
import jax.numpy as jnp
from jax.experimental import pallas as pl
from jax.experimental.pallas import tpu as pltpu

N = 10000
E = 160000
D = 256
H = 8
DH = D // H

ROW_BLK = 1000


def _pre_body(x_ref, g_ref, b_ref, ws_ref, bs_ref, wd_ref, bd_ref,
              h_ref, fs_lo_ref, fs_hi_ref, fd_lo_ref, fd_hi_ref):
    x = x_ref[...]
    mu = jnp.mean(x, axis=-1, keepdims=True)
    xc = x - mu
    var = jnp.mean(xc * xc, axis=-1, keepdims=True)
    h = xc * jax.lax.rsqrt(var + 1e-5) * g_ref[...] + b_ref[...]
    h_ref[...] = h
    fs = jnp.dot(h, ws_ref[...], preferred_element_type=jnp.float32) + bs_ref[...]
    fd = jnp.dot(h, wd_ref[...], preferred_element_type=jnp.float32) + bd_ref[...]
    fs_lo_ref[...] = fs[:, :128]
    fs_hi_ref[...] = fs[:, 128:]
    fd_lo_ref[...] = fd[:, :128]
    fd_hi_ref[...] = fd[:, 128:]


def _pre(features, ln_g, ln_b, W_src, b_src, W_dst, b_dst):
    grid = (N // ROW_BLK,)
    full2 = lambda shp: pl.BlockSpec(shp, lambda i: (0, 0))
    out_shapes = (
        jax.ShapeDtypeStruct((N, D), jnp.float32),
        jax.ShapeDtypeStruct((N, 128), jnp.float32),
        jax.ShapeDtypeStruct((N, 128), jnp.float32),
        jax.ShapeDtypeStruct((N, 128), jnp.float32),
        jax.ShapeDtypeStruct((N, 128), jnp.float32),
    )
    return pl.pallas_call(
        _pre_body,
        grid=grid,
        in_specs=[
            pl.BlockSpec((ROW_BLK, D), lambda i: (i, 0)),
            pl.BlockSpec((1, D), lambda i: (0, 0)),
            pl.BlockSpec((1, D), lambda i: (0, 0)),
            full2((D, D)),
            pl.BlockSpec((1, D), lambda i: (0, 0)),
            full2((D, D)),
            pl.BlockSpec((1, D), lambda i: (0, 0)),
        ],
        out_specs=(
            pl.BlockSpec((ROW_BLK, D), lambda i: (i, 0)),
            pl.BlockSpec((ROW_BLK, 128), lambda i: (i, 0)),
            pl.BlockSpec((ROW_BLK, 128), lambda i: (i, 0)),
            pl.BlockSpec((ROW_BLK, 128), lambda i: (i, 0)),
            pl.BlockSpec((ROW_BLK, 128), lambda i: (i, 0)),
        ),
        out_shape=out_shapes,
    )(features, ln_g.reshape(1, D), ln_b.reshape(1, D), W_src,
      b_src.reshape(1, D), W_dst, b_dst.reshape(1, D))


def _post_body(rst_ref, den_ref, h_ref, gb_ref, wo_ref, bo_ref, out_ref):
    rst = rst_ref[...] / (den_ref[...] + 1e-9)
    g = jnp.maximum(rst + h_ref[...] + gb_ref[...], 0.0)
    out_ref[...] = jnp.dot(g, wo_ref[...], preferred_element_type=jnp.float32) + bo_ref[...]


def _post(rst_unnorm, den_full, h, gat_bias, Wo, bo):
    grid = (N // ROW_BLK,)
    return pl.pallas_call(
        _post_body,
        grid=grid,
        in_specs=[
            pl.BlockSpec((ROW_BLK, D), lambda i: (i, 0)),
            pl.BlockSpec((ROW_BLK, D), lambda i: (i, 0)),
            pl.BlockSpec((ROW_BLK, D), lambda i: (i, 0)),
            pl.BlockSpec((1, D), lambda i: (0, 0)),
            pl.BlockSpec((D, D), lambda i: (0, 0)),
            pl.BlockSpec((1, D), lambda i: (0, 0)),
        ],
        out_specs=pl.BlockSpec((ROW_BLK, D), lambda i: (i, 0)),
        out_shape=jax.ShapeDtypeStruct((N, D), jnp.float32),
    )(rst_unnorm, den_full, h, gat_bias.reshape(1, D), Wo, bo.reshape(1, D))


def _edge_phase_jax(fs_lo, fs_hi, fd_lo, fd_hi, src, dst, attn):
    fs = jnp.concatenate([fs_lo, fs_hi], axis=1)
    fd = jnp.concatenate([fd_lo, fd_hi], axis=1)
    t = fs[src] + fd[dst]
    u = jnp.where(t > 0, t, 0.2 * t).reshape(E, H, DH)
    logits = jnp.einsum("ehd,hd->eh", u, attn)
    ex = jnp.exp(logits)
    msg = fs[src].reshape(E, H, DH) * ex[:, :, None]
    rst_unnorm = jax.ops.segment_sum(msg.reshape(E, D), dst, num_segments=N)
    den = jax.ops.segment_sum(ex, dst, num_segments=N)
    return rst_unnorm, den


def kernel(features, edge_index, ln_g, ln_b, W_src, b_src, W_dst, b_dst,
           attn, gat_bias, Wo, bo):
    h, fs_lo, fs_hi, fd_lo, fd_hi = _pre(
        features, ln_g, ln_b, W_src, b_src, W_dst, b_dst)
    src = edge_index[0]
    dst = edge_index[1]
    rst_unnorm, den = _edge_phase_jax(fs_lo, fs_hi, fd_lo, fd_hi, src, dst, attn)
    den_full = jnp.repeat(den, DH, axis=1)
    return _post(rst_unnorm, den_full, h, gat_bias, Wo, bo)


# speedup vs baseline: 16.2873x; 1.8837x over previous
import functools

import jax
import jax.numpy as jnp
from jax import lax
from jax.experimental import pallas as pl
from jax.experimental.pallas import tpu as pltpu
from jax.experimental.pallas import tpu_sc as plsc

N = 10000
E = 160000
D = 256
H = 8
DH = D // H

ROW_BLK = 1000

NC = 2
NS = 16
EPT = E // NS
K = 80
NCHUNK = EPT // K
NPAD = 10112
RPT = NPAD // NS
EGR = 1280


def _pre_body(x_ref, g_ref, b_ref, ws_ref, bs_ref, wd_ref, bd_ref,
              h_ref, fs_lo_ref, fs_hi_ref, fd_lo_ref, fd_hi_ref):
    x = x_ref[...]
    mu = jnp.mean(x, axis=-1, keepdims=True)
    xc = x - mu
    var = jnp.mean(xc * xc, axis=-1, keepdims=True)
    h = xc * jax.lax.rsqrt(var + 1e-5) * g_ref[...] + b_ref[...]
    h_ref[...] = h
    fs = jnp.dot(h, ws_ref[...], preferred_element_type=jnp.float32) + bs_ref[...]
    fd = jnp.dot(h, wd_ref[...], preferred_element_type=jnp.float32) + bd_ref[...]
    fs_lo_ref[...] = fs[:, :128]
    fs_hi_ref[...] = fs[:, 128:]
    fd_lo_ref[...] = fd[:, :128]
    fd_hi_ref[...] = fd[:, 128:]


def _pre(features, ln_g, ln_b, W_src, b_src, W_dst, b_dst):
    grid = (N // ROW_BLK,)
    full2 = lambda shp: pl.BlockSpec(shp, lambda i: (0, 0))
    out_shapes = (
        jax.ShapeDtypeStruct((N, D), jnp.float32),
        jax.ShapeDtypeStruct((N, 128), jnp.float32),
        jax.ShapeDtypeStruct((N, 128), jnp.float32),
        jax.ShapeDtypeStruct((N, 128), jnp.float32),
        jax.ShapeDtypeStruct((N, 128), jnp.float32),
    )
    return pl.pallas_call(
        _pre_body,
        grid=grid,
        in_specs=[
            pl.BlockSpec((ROW_BLK, D), lambda i: (i, 0)),
            pl.BlockSpec((1, D), lambda i: (0, 0)),
            pl.BlockSpec((1, D), lambda i: (0, 0)),
            full2((D, D)),
            pl.BlockSpec((1, D), lambda i: (0, 0)),
            full2((D, D)),
            pl.BlockSpec((1, D), lambda i: (0, 0)),
        ],
        out_specs=(
            pl.BlockSpec((ROW_BLK, D), lambda i: (i, 0)),
            pl.BlockSpec((ROW_BLK, 128), lambda i: (i, 0)),
            pl.BlockSpec((ROW_BLK, 128), lambda i: (i, 0)),
            pl.BlockSpec((ROW_BLK, 128), lambda i: (i, 0)),
            pl.BlockSpec((ROW_BLK, 128), lambda i: (i, 0)),
        ),
        out_shape=out_shapes,
    )(features, ln_g.reshape(1, D), ln_b.reshape(1, D), W_src,
      b_src.reshape(1, D), W_dst, b_dst.reshape(1, D))


def _post_body(rst_ref, den_ref, h_ref, gb_ref, wo_ref, bo_ref, out_ref):
    rst = rst_ref[...] / (den_ref[...] + 1e-9)
    g = jnp.maximum(rst + h_ref[...] + gb_ref[...], 0.0)
    out_ref[...] = jnp.dot(g, wo_ref[...], preferred_element_type=jnp.float32) + bo_ref[...]


def _post(rst_unnorm, den_full, h, gat_bias, Wo, bo):
    grid = (N // ROW_BLK,)
    return pl.pallas_call(
        _post_body,
        grid=grid,
        in_specs=[
            pl.BlockSpec((ROW_BLK, D), lambda i: (i, 0)),
            pl.BlockSpec((ROW_BLK, D), lambda i: (i, 0)),
            pl.BlockSpec((ROW_BLK, D), lambda i: (i, 0)),
            pl.BlockSpec((1, D), lambda i: (0, 0)),
            pl.BlockSpec((D, D), lambda i: (0, 0)),
            pl.BlockSpec((1, D), lambda i: (0, 0)),
        ],
        out_specs=pl.BlockSpec((ROW_BLK, D), lambda i: (i, 0)),
        out_shape=jax.ShapeDtypeStruct((N, D), jnp.float32),
    )(rst_unnorm, den_full, h, gat_bias.reshape(1, D), Wo, bo.reshape(1, D))


def _sc_edge_body(table, src_hbm, dst_hbm, attn_hbm,
                  rstA, rstB, exA, exB,
                  src_v, dst_v, gdst_v, rows_s, rows_d, exbuf, expack,
                  attn_v, rst_acc, sem1, sem2):
    c = lax.axis_index("c")
    s = lax.axis_index("s")
    base_r = pl.multiple_of(s * RPT, 8)
    offs = [0, 80, 160, 240, 320, 400, 480, 552]
    z16 = jnp.zeros((16,), jnp.float32)

    def zrow(i, carry):
        for j in range(8):
            rows_s[i, pl.ds(16 * j, 16)] = z16
        return carry

    lax.fori_loop(0, K, zrow, 0)
    for off in offs:
        ssl = pl.ds(pl.multiple_of(base_r + off, 8), K)
        pltpu.sync_copy(rows_s, rst_acc.at[ssl])
    pltpu.sync_copy(attn_hbm, attn_v)
    plsc.subcore_barrier()

    atn = [attn_v[c, pl.ds(16 * j, 16)] for j in range(8)]
    lane = lax.iota(jnp.int32, 16)
    perms = [(lane ^ (1 << b))[:, None] for b in range(4)]
    _dnums = lax.GatherDimensionNumbers(
        offset_dims=(), collapsed_slice_dims=(0,), start_index_map=(0,))

    def _lanesum(p):
        for pm in perms:
            p = p + lax.gather(p, pm, _dnums, slice_sizes=(1,),
                               mode=lax.GatherScatterMode.PROMISE_IN_BOUNDS)
        return p
    soff = c * N
    doff = 2 * N + c * N

    def edge_body(e, carry):
        sv = [rows_s[e, pl.ds(16 * j, 16)] for j in range(8)]
        w = []
        for j in range(8):
            t = sv[j] + rows_d[e, pl.ds(16 * j, 16)]
            u = jnp.maximum(t, 0.2 * t)
            w.append(u * atn[j])
        exv = jnp.zeros((16,), jnp.float32)
        for h in range(4):
            p = w[2 * h] + w[2 * h + 1]
            ev = jnp.exp(_lanesum(p))
            rows_s[e, pl.ds(32 * h, 16)] = sv[2 * h] * ev
            rows_s[e, pl.ds(32 * h + 16, 16)] = sv[2 * h + 1] * ev
            exv = exv + jnp.where(lane == h, ev, 0.0)
        exbuf[e, :] = exv
        return carry

    def do_chunk(k, row0):
        eb = pl.multiple_of(s * EPT + k * K, 8)
        pltpu.sync_copy(src_hbm.at[pl.ds(eb, K)], src_v)
        pltpu.sync_copy(dst_hbm.at[pl.ds(eb, K)], dst_v)
        for i in range(K // 16):
            sl = pl.ds(i * 16, 16)
            src_v[sl] = src_v[sl] + soff
            gdst_v[sl] = dst_v[sl] + doff
        cp1 = pltpu.async_copy(table.at[src_v], rows_s, sem1)
        cp2 = pltpu.async_copy(table.at[gdst_v], rows_d, sem2)
        cp1.wait()
        cp2.wait()
        lax.fori_loop(0, K, edge_body, 0)
        pltpu.sync_copy(rows_s, rst_acc.at[dst_v], add=True)

        def repack(r, carry):
            for m in range(8):
                expack[row0 + r, pl.ds(16 * m, 16)] = exbuf[8 * r + m, :]
            return carry

        lax.fori_loop(0, 10, repack, 0)

    def flush_ex(g):
        exsl = pl.ds(pl.multiple_of(s * EGR + 40 * g, 8), 40)

        @pl.when(c == 0)
        def _():
            pltpu.sync_copy(expack, exA.at[exsl])

        @pl.when(c == 1)
        def _():
            pltpu.sync_copy(expack, exB.at[exsl])

    def group_body(g, carry):
        for q in range(4):
            do_chunk(4 * g + q, 10 * q)
        flush_ex(g)
        return carry

    lax.fori_loop(0, NCHUNK // 4, group_body, 0)
    do_chunk(NCHUNK - 1, 0)
    flush_ex(NCHUNK // 4)
    plsc.subcore_barrier()


    def dump(rout):
        for off in offs:
            ssl = pl.ds(pl.multiple_of(base_r + off, 8), K)
            pltpu.sync_copy(rst_acc.at[ssl], rows_s)
            pltpu.sync_copy(rows_s, rout.at[ssl])

    @pl.when(c == 0)
    def _():
        dump(rstA)

    @pl.when(c == 1)
    def _():
        dump(rstB)


def _sc_edge(table, src, dst, attn2):
    mesh = plsc.VectorSubcoreMesh(
        core_axis_name="c", subcore_axis_name="s", num_cores=NC,
        num_subcores=NS)
    f = pl.kernel(
        _sc_edge_body,
        out_type=(
            jax.ShapeDtypeStruct((NPAD, 128), jnp.float32),
            jax.ShapeDtypeStruct((NPAD, 128), jnp.float32),
            jax.ShapeDtypeStruct((NS * EGR, 128), jnp.float32),
            jax.ShapeDtypeStruct((NS * EGR, 128), jnp.float32),
        ),
        mesh=mesh,
        scratch_types=[
            pltpu.VMEM((K,), jnp.int32),
            pltpu.VMEM((K,), jnp.int32),
            pltpu.VMEM((K,), jnp.int32),
            pltpu.VMEM((K, 128), jnp.float32),
            pltpu.VMEM((K, 128), jnp.float32),
            pltpu.VMEM((K, 16), jnp.float32),
            pltpu.VMEM((40, 128), jnp.float32),
            pltpu.VMEM((NC, 128), jnp.float32),
            pltpu.VMEM_SHARED((NPAD, 128), jnp.float32),
            pltpu.SemaphoreType.DMA,
            pltpu.SemaphoreType.DMA,
        ],
    )
    return f(table, src, dst, attn2)


def kernel(features, edge_index, ln_g, ln_b, W_src, b_src, W_dst, b_dst,
           attn, gat_bias, Wo, bo):
    h, fs_lo, fs_hi, fd_lo, fd_hi = _pre(
        features, ln_g, ln_b, W_src, b_src, W_dst, b_dst)
    src = edge_index[0]
    dst = edge_index[1]
    table = jnp.concatenate([fs_lo, fs_hi, fd_lo, fd_hi], axis=0)
    attn2 = attn.reshape(NC, 128)
    rstA, rstB, exA, exB = _sc_edge(table, src, dst, attn2)
    rst_unnorm = jnp.concatenate([rstA[:N], rstB[:N]], axis=1)
    exa = exA.reshape(NS, EGR, 128)[:, :EPT // 8].reshape(E, 16)[:, :4]
    exb = exB.reshape(NS, EGR, 128)[:, :EPT // 8].reshape(E, 16)[:, :4]
    ex8 = jnp.concatenate([exa, exb], axis=1)
    den = jax.ops.segment_sum(ex8, dst, num_segments=N)
    den_full = jnp.repeat(den, DH, axis=1)
    return _post(rst_unnorm, den_full, h, gat_bias, Wo, bo)
